# Initial kernel scaffold; baseline (speedup 1.0000x reference)
import jax
import jax.numpy as jnp
from jax.experimental import pallas as pl


def kernel(x, tables, W):
    raise NotImplementedError("write your pallas kernel here")


# speedup vs baseline: 2.9806x; 2.9806x over previous
import functools

import jax
import jax.numpy as jnp
from jax import lax
from jax.experimental import pallas as pl
from jax.experimental.pallas import tpu as pltpu
from jax.experimental.pallas import tpu_sc as plsc

N_CAT = 26
N_NUM = 4
VOCAB = 100000
EMB_DIM = 32
B, L = 4096, 50
N_TOK = B * L
N_SLOT = N_CAT + 1
NF = N_CAT + N_NUM
OUT_DIM = N_SLOT * EMB_DIM

NC, NS = 2, 16
NW = NC * NS
TOK_PER_W = N_TOK // NW
C = 64
ROWS = C * N_SLOT
G = 14
ROWS_PAD = G * 128
NCH = TOK_PER_W // C


def _body(x_hbm, tab_hbm, wt_hbm, out_hbm,
          x_a, x_b, idx_a, idx_b, out_a, out_b, wt_v,
          gsem_a, gsem_b, wsem_a, wsem_b, xsem_a, xsem_b):
    xv = (x_a, x_b)
    idxv = (idx_a, idx_b)
    outv = (out_a, out_b)
    gsem = (gsem_a, gsem_b)
    wsem = (wsem_a, wsem_b)
    xsem = (xsem_a, xsem_b)

    wid = lax.axis_index("s") * NC + lax.axis_index("c")
    tok_w = wid * TOK_PER_W

    pltpu.sync_copy(wt_hbm, wt_v)
    w_lo = [wt_v[k, pl.ds(0, 16)] for k in range(N_NUM)]
    w_hi = [wt_v[k, pl.ds(16, 16)] for k in range(N_NUM)]

    zeros16 = jnp.zeros((16,), jnp.int32)
    for p in range(2):
        for g in range((ROWS_PAD - ROWS) // 16):
            idxv[p][pl.ds(ROWS + g * 16, 16)] = zeros16

    lane = lax.iota(jnp.int32, 16)

    def tok0_of(c):
        return tok_w + c * C

    def pre_x(c, p):
        pltpu.async_copy(x_hbm.at[pl.ds(tok0_of(c) * NF, C * NF)],
                         xv[p], xsem[p])

    def wait_x(p):
        pltpu.make_async_copy(x_hbm.at[pl.ds(0, C * NF)], xv[p],
                              xsem[p]).wait()

    def build_idx(p):
        for f in range(N_CAT):
            for g in range(C // 16):
                rows = g * 16 + lane
                vals = plsc.load_gather(xv[p], [rows * NF + f]) + f * VOCAB
                plsc.store_scatter(idxv[p], [rows * N_SLOT + f], vals)
        for g in range(C // 16):
            rows = g * 16 + lane
            plsc.store_scatter(idxv[p], [rows * N_SLOT + N_CAT], zeros16)

    def fire_gathers(p):
        for g in range(G):
            pltpu.async_copy(tab_hbm.at[idxv[p].at[pl.ds(g * 128, 128)]],
                             outv[p].at[pl.ds(g * 128, 128)], gsem[p])

    def drain_gathers(p):
        for g in range(G):
            pltpu.make_async_copy(
                tab_hbm.at[idxv[p].at[pl.ds(g * 128, 128)]],
                outv[p].at[pl.ds(g * 128, 128)], gsem[p]).wait()

    def numeric(p):
        def tok(t, tc):
            acc_lo = jnp.zeros((16,), jnp.float32)
            acc_hi = jnp.zeros((16,), jnp.float32)
            xt = xv[p][pl.ds(t * NF + NF - 16, 16)].astype(jnp.float32)
            for k in range(N_NUM):
                s = xt[16 - N_NUM + k]
                acc_lo = acc_lo + s * w_lo[k]
                acc_hi = acc_hi + s * w_hi[k]
            row = t * N_SLOT + N_CAT
            outv[p][row, pl.ds(0, 16)] = acc_lo
            outv[p][row, pl.ds(16, 16)] = acc_hi
            return tc

        lax.fori_loop(0, C, tok, 0, unroll=4)

    def fire_wb(c, p):
        pltpu.async_copy(outv[p].at[pl.ds(0, ROWS)],
                         out_hbm.at[pl.ds(tok0_of(c) * N_SLOT, ROWS)],
                         wsem[p])

    def wait_wb(p):
        pltpu.make_async_copy(outv[p].at[pl.ds(0, ROWS)],
                              out_hbm.at[pl.ds(0, ROWS)], wsem[p]).wait()

    def half(c, p, *, first=False, warm=False, last=False):
        wait_x(p)
        build_idx(p)
        if not (first or warm):
            wait_wb(p)
        fire_gathers(p)
        if not first:
            drain_gathers(1 - p)
            numeric(1 - p)
            fire_wb(c - 1, 1 - p)
        if not last:
            pre_x(c + 1, 1 - p)

    pre_x(0, 0)
    half(0, 0, first=True)
    half(1, 1, warm=True)
    half(2, 0)

    def pair(i, carry):
        c = 3 + 2 * i
        half(c, 1)
        half(c + 1, 0)
        return carry

    lax.fori_loop(0, (NCH - 4) // 2, pair, 0)

    half(NCH - 1, 1, last=True)
    drain_gathers(1)
    numeric(1)
    fire_wb(NCH - 1, 1)
    wait_wb(0)
    wait_wb(1)


@functools.partial(
    pl.kernel,
    out_type=jax.ShapeDtypeStruct((N_TOK * N_SLOT, EMB_DIM), jnp.float32),
    mesh=plsc.VectorSubcoreMesh(core_axis_name="c", subcore_axis_name="s"),
    compiler_params=pltpu.CompilerParams(
        needs_layout_passes=False, use_tc_tiling_on_sc=False),
    scratch_types=[
        pltpu.VMEM((C * NF,), jnp.int32),
        pltpu.VMEM((C * NF,), jnp.int32),
        pltpu.VMEM((ROWS_PAD,), jnp.int32),
        pltpu.VMEM((ROWS_PAD,), jnp.int32),
        pltpu.VMEM((ROWS_PAD, EMB_DIM), jnp.float32),
        pltpu.VMEM((ROWS_PAD, EMB_DIM), jnp.float32),
        pltpu.VMEM((N_NUM, EMB_DIM), jnp.float32),
        pltpu.SemaphoreType.DMA,
        pltpu.SemaphoreType.DMA,
        pltpu.SemaphoreType.DMA,
        pltpu.SemaphoreType.DMA,
        pltpu.SemaphoreType.DMA,
        pltpu.SemaphoreType.DMA,
    ],
)
def _sc_embed(x_hbm, tab_hbm, wt_hbm, out_hbm, *scratch):
    _body(x_hbm, tab_hbm, wt_hbm, out_hbm, *scratch)


def kernel(x, tables, W):
    x2 = x.reshape(N_TOK * NF).astype(jnp.int32)
    tab = tables.reshape(N_CAT * VOCAB, EMB_DIM)
    wt = W.T.astype(jnp.float32)
    out = _sc_embed(x2, tab, wt)
    return out.reshape(B, L, OUT_DIM)
